# Initial kernel scaffold; baseline (speedup 1.0000x reference)
import jax
import jax.numpy as jnp
from jax.experimental import pallas as pl


def kernel(x, adj, W1, b1, W2, b2):
    raise NotImplementedError("write your pallas kernel here")


# speedup vs baseline: 1.1277x; 1.1277x over previous
import functools

import jax
import jax.numpy as jnp
from jax.experimental import pallas as pl
from jax.experimental.pallas import tpu as pltpu

_N = 4096
_GRID = 8
_BLK = _N // _GRID


def _gcn_body(x_ref, adj_ref, w1_ref, b1_ref, w2_ref, b2_ref, out_ref,
              adjb_ref, s_ref, h_ref, g_ref):
    p = pl.program_id(0)
    i = pl.program_id(1)

    @pl.when(p == 0)
    def _phase0():
        @pl.when(i == 0)
        def _support():
            s_ref[...] = jnp.dot(
                x_ref[...], w1_ref[...],
                preferred_element_type=jnp.float32).astype(jnp.bfloat16)

        a = adj_ref[...].astype(jnp.bfloat16)
        adjb_ref[pl.ds(i * _BLK, _BLK), :] = a
        hpre = jnp.dot(a, s_ref[...],
                       preferred_element_type=jnp.float32) + b1_ref[...]
        h_ref[pl.ds(i * _BLK, _BLK), :] = jnp.maximum(hpre, 0.0).astype(
            jnp.bfloat16)

    @pl.when(p == 1)
    def _phase1():
        @pl.when(i == 0)
        def _support2():
            g_ref[...] = jnp.dot(
                h_ref[...], w2_ref[...],
                preferred_element_type=jnp.float32).astype(jnp.bfloat16)

        o = jnp.dot(adjb_ref[pl.ds(i * _BLK, _BLK), :], g_ref[...],
                    preferred_element_type=jnp.float32) + b2_ref[...]
        m = jnp.max(o, axis=1, keepdims=True)
        e = o - m
        lse = jnp.log(jnp.sum(jnp.exp(e), axis=1, keepdims=True))
        out_ref[...] = e - lse


@functools.partial(jax.jit, static_argnames=())
def kernel(x, adj, W1, b1, W2, b2):
    n, d_in = x.shape
    d_h = W1.shape[1]
    d_out = W2.shape[1]
    b1r = b1.reshape(1, d_h)
    b2r = b2.reshape(1, d_out)

    grid = (2, _GRID)
    out = pl.pallas_call(
        _gcn_body,
        grid=grid,
        in_specs=[
            pl.BlockSpec((n, d_in), lambda p, i: (0, 0)),
            pl.BlockSpec((_BLK, n),
                         lambda p, i: (jnp.where(p == 0, i, _GRID - 1), 0)),
            pl.BlockSpec((d_in, d_h), lambda p, i: (0, 0)),
            pl.BlockSpec((1, d_h), lambda p, i: (0, 0)),
            pl.BlockSpec((d_h, d_out), lambda p, i: (0, 0)),
            pl.BlockSpec((1, d_out), lambda p, i: (0, 0)),
        ],
        out_specs=pl.BlockSpec((_BLK, d_out),
                               lambda p, i: (jnp.where(p == 1, i, 0), 0)),
        out_shape=jax.ShapeDtypeStruct((n, d_out), jnp.float32),
        scratch_shapes=[
            pltpu.VMEM((n, n), jnp.bfloat16),
            pltpu.VMEM((n, d_h), jnp.bfloat16),
            pltpu.VMEM((n, d_h), jnp.bfloat16),
            pltpu.VMEM((n, d_out), jnp.bfloat16),
        ],
        compiler_params=pltpu.CompilerParams(
            vmem_limit_bytes=100 * 1024 * 1024,
        ),
    )(x, adj, W1, b1r, W2, b2r)
    return out
